# baseline (speedup 1.0000x reference)
import functools

import jax
import jax.numpy as jnp
from jax import lax
from jax.experimental import pallas as pl
from jax.experimental.pallas import tpu as pltpu
from jax.experimental.pallas import tpu_sc as plsc

B = 16384
F = 26
V = 100000
D = 32
C = 16

NC, NS = 2, 16
NW = NC * NS
FD = F * D
PAIRS_W = FD // NW
QUART = B // 4
LANES = 16
NCHAIN = 8


def _make_gather(nfd, fd0):
    ppw = nfd // NW
    mesh = plsc.VectorSubcoreMesh(core_axis_name="c", subcore_axis_name="s")

    @functools.partial(
        pl.kernel,
        mesh=mesh,
        compiler_params=pltpu.CompilerParams(
            use_tc_tiling_on_sc=True, needs_layout_passes=False),
        out_type=jax.ShapeDtypeStruct((nfd, B), jnp.float32),
        scratch_types=[
            pltpu.VMEM((V,), jnp.float32),
            pltpu.VMEM((B,), jnp.int32),
            pltpu.VMEM((QUART,), jnp.float32),
            pltpu.VMEM((QUART,), jnp.float32),
            pltpu.SemaphoreType.DMA,
            pltpu.SemaphoreType.DMA,
            pltpu.SemaphoreType.DMA,
        ],
    )
    def gather_kernel(tbl_hbm, idx_hbm, out_hbm,
                      col_v, idx_v, g0, g1, sem_c, sem_w0, sem_w1):
        wid = lax.axis_index("s") * NC + lax.axis_index("c")
        bufs = (g0, g1)
        sems = (sem_w0, sem_w1)

        def pair_body(p, prev_f):
            fd = wid * ppw + p
            f = (fd0 + fd) // D
            ccol = pltpu.make_async_copy(
                tbl_hbm.at[f, (fd0 + fd) % D], col_v, sem_c)
            ccol.start()

            @pl.when(f != prev_f)
            def _():
                pltpu.sync_copy(idx_hbm.at[f], idx_v)

            ccol.wait()

            for q in range(4):
                buf, sem = bufs[q % 2], sems[q % 2]

                @pl.when(p * 4 + q >= 2)
                def _():
                    pltpu.make_async_copy(
                        buf, out_hbm.at[fd, pl.ds(0, QUART)], sem).wait()

                base = q * QUART
                blk = LANES * NCHAIN

                def gather_blk(j, carry, base=base, buf=buf):
                    o = j * blk
                    ivs = [idx_v[pl.ds(base + o + k * LANES, LANES)]
                           for k in range(NCHAIN)]
                    vals = [plsc.load_gather(col_v, [iv]) for iv in ivs]
                    for k in range(NCHAIN):
                        buf[pl.ds(o + k * LANES, LANES)] = vals[k]
                    return carry

                lax.fori_loop(0, QUART // blk, gather_blk, 0, unroll=2)
                pltpu.make_async_copy(
                    buf, out_hbm.at[fd, pl.ds(base, QUART)], sem).start()
            return f

        lax.fori_loop(0, ppw, pair_body, -1)
        pltpu.make_async_copy(g0, out_hbm.at[0, pl.ds(0, QUART)], sem_w0).wait()
        pltpu.make_async_copy(g1, out_hbm.at[0, pl.ds(0, QUART)], sem_w1).wait()

    return gather_kernel


FDA = 704
FDB = FD - FDA


@functools.cache
def _gather_a():
    return _make_gather(FDA, 0)


@functools.cache
def _gather_b():
    return _make_gather(FDB, FDA)


BB = 2048


def _partial_kernel(x_ref, w_ref, o_ref):
    o_ref[...] = lax.dot_general(w_ref[...], x_ref[...],
                                 (((1,), (0,)), ((), ())),
                                 preferred_element_type=jnp.float32)


def _partial(emb_a, w1a):
    return pl.pallas_call(
        _partial_kernel,
        grid=(B // BB,),
        in_specs=[
            pl.BlockSpec((FDA, BB), lambda i: (0, i)),
            pl.BlockSpec((D, FDA), lambda i: (0, 0)),
        ],
        out_specs=pl.BlockSpec((D, BB), lambda i: (0, i)),
        out_shape=jax.ShapeDtypeStruct((D, B), jnp.float32),
    )(emb_a, w1a)


def _final_kernel(p_ref, x_ref, c_ref, w1_ref, w2_ref, b_ref, o_ref):
    acc = p_ref[...] + lax.dot_general(w1_ref[...], x_ref[...],
                                       (((1,), (0,)), ((), ())),
                                       preferred_element_type=jnp.float32)
    acc += lax.dot_general(w2_ref[...], c_ref[...],
                           (((1,), (0,)), ((), ())),
                           preferred_element_type=jnp.float32)
    acc += b_ref[...]
    o_ref[...] = jnp.where(acc >= 0, acc, 0.5 * acc)


def _final(part, emb_b, cont_t, w1b, w2, bias_col):
    return pl.pallas_call(
        _final_kernel,
        grid=(B // BB,),
        in_specs=[
            pl.BlockSpec((D, BB), lambda i: (0, i)),
            pl.BlockSpec((FDB, BB), lambda i: (0, i)),
            pl.BlockSpec((C, BB), lambda i: (0, i)),
            pl.BlockSpec((D, FDB), lambda i: (0, 0)),
            pl.BlockSpec((D, C), lambda i: (0, 0)),
            pl.BlockSpec((D, 1), lambda i: (0, 0)),
        ],
        out_specs=pl.BlockSpec((D, BB), lambda i: (0, i)),
        out_shape=jax.ShapeDtypeStruct((D, B), jnp.float32),
    )(part, emb_b, cont_t, w1b, w2, bias_col)


def kernel(cat_indices, cont, tables, W, b):
    tables_t = jnp.transpose(tables, (0, 2, 1))
    idx_t = jnp.transpose(cat_indices, (1, 0))
    cont_t = jnp.transpose(cont, (1, 0))
    emb_a = _gather_a()(tables_t, idx_t)
    emb_b = _gather_b()(tables_t, idx_t)
    part = _partial(emb_a, W[:, :FDA])
    out_t = _final(part, emb_b, cont_t,
                   W[:, FDA:FD], W[:, FD:], b[:, None])
    return jnp.transpose(out_t, (1, 0))


# speedup vs baseline: 1.0235x; 1.0235x over previous
import functools

import jax
import jax.numpy as jnp
from jax import lax
from jax.experimental import pallas as pl
from jax.experimental.pallas import tpu as pltpu
from jax.experimental.pallas import tpu_sc as plsc

B = 16384
F = 26
V = 100000
D = 32
C = 16

NC, NS = 2, 16
NW = NC * NS
FD = F * D
PAIRS_W = FD // NW
QUART = B // 4
LANES = 16
NCHAIN = 8


def _make_gather():
    mesh = plsc.VectorSubcoreMesh(core_axis_name="c", subcore_axis_name="s")

    @functools.partial(
        pl.kernel,
        mesh=mesh,
        compiler_params=pltpu.CompilerParams(
            use_tc_tiling_on_sc=True, needs_layout_passes=False),
        out_type=jax.ShapeDtypeStruct((FD, B), jnp.float32),
        scratch_types=[
            pltpu.VMEM((V,), jnp.float32),
            pltpu.VMEM((B,), jnp.int32),
            pltpu.VMEM((QUART,), jnp.float32),
            pltpu.VMEM((QUART,), jnp.float32),
            pltpu.SemaphoreType.DMA,
            pltpu.SemaphoreType.DMA,
            pltpu.SemaphoreType.DMA,
        ],
    )
    def gather_kernel(tbl_hbm, idx_hbm, out_hbm,
                      col_v, idx_v, g0, g1, sem_c, sem_w0, sem_w1):
        wid = lax.axis_index("s") * NC + lax.axis_index("c")
        bufs = (g0, g1)
        sems = (sem_w0, sem_w1)

        def pair_body(p, prev_f):
            fd = wid * PAIRS_W + p
            f = fd // D
            ccol = pltpu.make_async_copy(tbl_hbm.at[f, fd % D], col_v, sem_c)
            ccol.start()

            @pl.when(f != prev_f)
            def _():
                pltpu.sync_copy(idx_hbm.at[f], idx_v)

            ccol.wait()

            for q in range(4):
                buf, sem = bufs[q % 2], sems[q % 2]

                @pl.when(p * 4 + q >= 2)
                def _():
                    pltpu.make_async_copy(
                        buf, out_hbm.at[fd, pl.ds(0, QUART)], sem).wait()

                base = q * QUART
                blk = LANES * NCHAIN

                def gather_blk(j, carry, base=base, buf=buf):
                    o = j * blk
                    ivs = [idx_v[pl.ds(base + o + k * LANES, LANES)]
                           for k in range(NCHAIN)]
                    vals = [plsc.load_gather(col_v, [iv]) for iv in ivs]
                    for k in range(NCHAIN):
                        buf[pl.ds(o + k * LANES, LANES)] = vals[k]
                    return carry

                lax.fori_loop(0, QUART // blk, gather_blk, 0, unroll=2)
                pltpu.make_async_copy(
                    buf, out_hbm.at[fd, pl.ds(base, QUART)], sem).start()
            return f

        lax.fori_loop(0, PAIRS_W, pair_body, -1)
        pltpu.make_async_copy(g0, out_hbm.at[0, pl.ds(0, QUART)], sem_w0).wait()
        pltpu.make_async_copy(g1, out_hbm.at[0, pl.ds(0, QUART)], sem_w1).wait()

    return gather_kernel


@functools.cache
def _gather_fn():
    return _make_gather()


BB = 2048


def _combine_kernel(x_ref, c_ref, w1_ref, w2_ref, b_ref, o_ref):
    acc = lax.dot_general(w1_ref[...], x_ref[...],
                          (((1,), (0,)), ((), ())),
                          preferred_element_type=jnp.float32)
    acc += lax.dot_general(w2_ref[...], c_ref[...],
                           (((1,), (0,)), ((), ())),
                           preferred_element_type=jnp.float32)
    acc += b_ref[...]
    o_ref[...] = jnp.where(acc >= 0, acc, 0.5 * acc)


def _combine(emb_t, cont_t, w1, w2, bias_col):
    return pl.pallas_call(
        _combine_kernel,
        grid=(B // BB,),
        in_specs=[
            pl.BlockSpec((FD, BB), lambda i: (0, i)),
            pl.BlockSpec((C, BB), lambda i: (0, i)),
            pl.BlockSpec((D, FD), lambda i: (0, 0)),
            pl.BlockSpec((D, C), lambda i: (0, 0)),
            pl.BlockSpec((D, 1), lambda i: (0, 0)),
        ],
        out_specs=pl.BlockSpec((D, BB), lambda i: (0, i)),
        out_shape=jax.ShapeDtypeStruct((D, B), jnp.float32),
    )(emb_t, cont_t, w1, w2, bias_col)


def kernel(cat_indices, cont, tables, W, b):
    tables_t = jnp.transpose(tables, (0, 2, 1))
    idx_t = jnp.transpose(cat_indices, (1, 0))
    cont_t = jnp.transpose(cont, (1, 0))
    emb_t = _gather_fn()(tables_t, idx_t)
    w1 = W[:, :FD]
    w2 = W[:, FD:]
    out_t = _combine(emb_t, cont_t, w1, w2, b[:, None])
    return jnp.transpose(out_t, (1, 0))
